# Initial kernel scaffold; baseline (speedup 1.0000x reference)
import jax
import jax.numpy as jnp
from jax.experimental import pallas as pl


def kernel(x, edge_index, W, b):
    raise NotImplementedError("write your pallas kernel here")


# speedup vs baseline: 24.2129x; 24.2129x over previous
import functools

import jax
import jax.numpy as jnp
from jax import lax
from jax.experimental import pallas as pl
from jax.experimental.pallas import tpu as pltpu
from jax.experimental.pallas import tpu_sc as plsc

NC = 2
NS = 16
L = 16
K = 80


def _fill_iota_idx(ibuf, row, base):
    for v in range(K // L):
        ibuf[row, pl.ds(v * L, L)] = base + v * L + lax.iota(jnp.int32, L)


def _deg_body(npad, ept, dst_hbm, cnt_hbm, ibuf, vbuf, rowbuf,
              sem, deg_sh):
    c = lax.axis_index("c")
    s = lax.axis_index("s")
    rpt = npad // NS
    row0 = s * rpt

    def zrow(r, carry):
        vbuf[r, :] = jnp.zeros((L,), jnp.float32)
        return carry

    lax.fori_loop(0, K, zrow, 0)

    def iinit(t, carry):
        _fill_iota_idx(ibuf, 0, row0 + t * K)
        pltpu.sync_copy(vbuf, deg_sh.at[ibuf.at[0]])
        return carry

    lax.fori_loop(0, rpt // K, iinit, 0)

    def orow(r, carry):
        vbuf[r, :] = jnp.full((L,), 1.0, jnp.float32)
        return carry

    lax.fori_loop(0, K, orow, 0)
    plsc.subcore_barrier()

    eb = (c * NS + s) * ept

    def chunk(ch, carry):
        pltpu.sync_copy(dst_hbm.at[pl.ds(eb + ch * K, K)], ibuf.at[0])
        pltpu.sync_copy(vbuf, deg_sh.at[ibuf.at[0]], add=True)
        return carry

    lax.fori_loop(0, ept // K, chunk, 0)
    plsc.subcore_barrier()

    def rback(t, carry):
        _fill_iota_idx(ibuf, 0, row0 + t * K)
        pltpu.async_copy(deg_sh.at[ibuf.at[0]], rowbuf, sem).wait()
        pltpu.sync_copy(rowbuf,
                        cnt_hbm.at[pl.ds(c * npad + row0 + t * K, K)])
        return carry

    lax.fori_loop(0, rpt // K, rback, 0)


def _scat_body(npad, d, ept, hp_hbm, src_hbm, dst_hbm, acc_hbm,
               sbuf0, sbuf1, dbuf0, dbuf1, ibuf, zrows, rows, sem0, sem1,
               gsem, acc_sh):
    c = lax.axis_index("c")
    s = lax.axis_index("s")
    rpt = npad // NS
    row0 = s * rpt

    def zrow(r, carry):
        for j in range(d // L):
            zrows[r, pl.ds(j * L, L)] = jnp.zeros((L,), jnp.float32)
        return carry

    lax.fori_loop(0, K, zrow, 0)

    def iinit(t, carry):
        _fill_iota_idx(ibuf, 0, row0 + t * K)
        pltpu.sync_copy(zrows, acc_sh.at[ibuf.at[0]])
        return carry

    lax.fori_loop(0, rpt // K, iinit, 0)
    plsc.subcore_barrier()

    eb = (c * NS + s) * ept
    cpt = ept // K
    sems = (sem0, sem1)
    sbufs = (sbuf0, sbuf1)
    dbufs = (dbuf0, dbuf1)
    pltpu.sync_copy(src_hbm.at[pl.ds(eb, K)], sbuf0.at[0])
    pltpu.async_copy(hp_hbm.at[sbuf0.at[0]], rows.at[0], sem0)

    def step(k, carry):
        for j in range(2):
            ch = 2 * k + j
            pltpu.sync_copy(src_hbm.at[pl.ds(eb + (ch + 1) * K, K)],
                            sbufs[1 - j].at[0])
            pltpu.async_copy(hp_hbm.at[sbufs[1 - j].at[0]], rows.at[1 - j],
                             sems[1 - j])
            pltpu.sync_copy(dst_hbm.at[pl.ds(eb + ch * K, K)],
                            dbufs[j].at[0])
            pltpu.make_async_copy(hp_hbm.at[sbufs[j].at[0]], rows.at[j],
                                  sems[j]).wait()
            pltpu.sync_copy(rows.at[j], acc_sh.at[dbufs[j].at[0]], add=True)
        return carry

    lax.fori_loop(0, (cpt - 1) // 2, step, 0)
    last = cpt - 1
    pltpu.sync_copy(dst_hbm.at[pl.ds(eb + last * K, K)], dbuf0.at[0])
    pltpu.make_async_copy(hp_hbm.at[sbuf0.at[0]], rows.at[0], sem0).wait()
    pltpu.sync_copy(rows.at[0], acc_sh.at[dbuf0.at[0]], add=True)

    plsc.subcore_barrier()

    def rback(t, carry):
        _fill_iota_idx(ibuf, 0, row0 + t * K)
        pltpu.async_copy(acc_sh.at[ibuf.at[0]], zrows, gsem).wait()
        pltpu.sync_copy(zrows,
                        acc_hbm.at[pl.ds(c * npad + row0 + t * K, K)])
        return carry

    lax.fori_loop(0, rpt // K, rback, 0)


def _mm_body(x_ref, w_ref, cnt0_ref, cnt1_ref, hp_ref):
    deg = 1.0 + cnt0_ref[:, 0:1] + cnt1_ref[:, 0:1]
    hp_ref[...] = jnp.dot(x_ref[...], w_ref[...],
                          preferred_element_type=jnp.float32) * lax.rsqrt(deg)


def _fin_body(acc0_ref, acc1_ref, hp_ref, cnt0_ref, cnt1_ref, b_ref,
              out_ref):
    deg = 1.0 + cnt0_ref[:, 0:1] + cnt1_ref[:, 0:1]
    out_ref[...] = (acc0_ref[...] + acc1_ref[...] + hp_ref[...]) \
        * lax.rsqrt(deg) + b_ref[...]


def kernel(x, edge_index, W, b):
    n, d_in = x.shape
    d_out = W.shape[1]
    e = edge_index.shape[1]
    nw = NC * NS
    ept = e // nw
    R = 1024
    npad = ((n + R - 1) // R) * R

    src = edge_index[0]
    dst = edge_index[1]

    mesh = plsc.VectorSubcoreMesh(core_axis_name="c", subcore_axis_name="s",
                                  num_cores=NC, num_subcores=NS)

    deg_call = pl.kernel(
        functools.partial(_deg_body, npad, ept),
        out_type=jax.ShapeDtypeStruct((NC * npad, L), jnp.float32),
        mesh=mesh,
        scratch_types=[
            pltpu.VMEM((1, K), jnp.int32),
            pltpu.VMEM((K, L), jnp.float32),
            pltpu.VMEM((K, L), jnp.float32),
            pltpu.SemaphoreType.DMA,
            pltpu.VMEM_SHARED((npad, L), jnp.float32),
        ],
    )
    cnt = deg_call(dst)
    cnt0 = cnt[:npad]
    cnt1 = cnt[npad:]

    grid = (npad // R,)
    blk = lambda i: (i, 0)
    hp = pl.pallas_call(
        _mm_body,
        grid=grid,
        in_specs=[
            pl.BlockSpec((R, d_in), blk),
            pl.BlockSpec((d_in, d_out), lambda i: (0, 0)),
            pl.BlockSpec((R, L), blk),
            pl.BlockSpec((R, L), blk),
        ],
        out_specs=pl.BlockSpec((R, d_out), blk),
        out_shape=jax.ShapeDtypeStruct((n, d_out), jnp.float32),
    )(x, W, cnt0, cnt1)

    scat_call = pl.kernel(
        functools.partial(_scat_body, npad, d_out, ept),
        out_type=jax.ShapeDtypeStruct((NC * npad, d_out), jnp.float32),
        mesh=mesh,
        scratch_types=[
            pltpu.VMEM((1, K), jnp.int32),
            pltpu.VMEM((1, K), jnp.int32),
            pltpu.VMEM((1, K), jnp.int32),
            pltpu.VMEM((1, K), jnp.int32),
            pltpu.VMEM((1, K), jnp.int32),
            pltpu.VMEM((K, d_out), jnp.float32),
            pltpu.VMEM((2, K, d_out), jnp.float32),
            pltpu.SemaphoreType.DMA,
            pltpu.SemaphoreType.DMA,
            pltpu.SemaphoreType.DMA,
            pltpu.VMEM_SHARED((npad, d_out), jnp.float32),
        ],
    )
    acc = scat_call(hp, src, dst)

    out = pl.pallas_call(
        _fin_body,
        grid=grid,
        in_specs=[
            pl.BlockSpec((R, d_out), blk),
            pl.BlockSpec((R, d_out), blk),
            pl.BlockSpec((R, d_out), blk),
            pl.BlockSpec((R, L), blk),
            pl.BlockSpec((R, L), blk),
            pl.BlockSpec((1, d_out), lambda i: (0, 0)),
        ],
        out_specs=pl.BlockSpec((R, d_out), blk),
        out_shape=jax.ShapeDtypeStruct((n, d_out), jnp.float32),
    )(acc[:npad], acc[npad:], hp, cnt0, cnt1, b.reshape(1, d_out))

    return out


# speedup vs baseline: 35.2527x; 1.4559x over previous
import functools

import jax
import jax.numpy as jnp
from jax import lax
from jax.experimental import pallas as pl
from jax.experimental.pallas import tpu as pltpu
from jax.experimental.pallas import tpu_sc as plsc

NC = 2
NS = 16
L = 16
K = 80
NB = 25


def _fill_iota_idx(ibuf, row, base):
    for v in range(K // L):
        ibuf[row, pl.ds(v * L, L)] = base + v * L + lax.iota(jnp.int32, L)


def _deg_body(npad, ept, dst_hbm, cnt_hbm, ibuf, dibuf, vbuf, rowbuf,
              sem, deg_sh):
    c = lax.axis_index("c")
    s = lax.axis_index("s")
    rpt = npad // NS
    row0 = s * rpt

    def zrow(r, carry):
        vbuf[r, :] = jnp.zeros((L,), jnp.float32)
        return carry

    lax.fori_loop(0, K, zrow, 0)

    def iinit(t, carry):
        _fill_iota_idx(ibuf, 0, row0 + t * K)
        pltpu.sync_copy(vbuf, deg_sh.at[ibuf.at[0]])
        return carry

    lax.fori_loop(0, rpt // K, iinit, 0)

    def orow(r, carry):
        vbuf[r, :] = jnp.full((L,), 1.0, jnp.float32)
        return carry

    lax.fori_loop(0, K, orow, 0)
    plsc.subcore_barrier()

    w = c * NS + s

    def dbatch(b, carry):
        pltpu.sync_copy(dst_hbm.at[w, b], dibuf)

        def chunk(ch, c2):
            pltpu.sync_copy(vbuf, deg_sh.at[dibuf.at[ch]], add=True)
            return c2

        lax.fori_loop(0, NB, chunk, 0)
        return carry

    lax.fori_loop(0, (ept // K) // NB, dbatch, 0)
    plsc.subcore_barrier()

    def rback(t, carry):
        _fill_iota_idx(ibuf, 0, row0 + t * K)
        pltpu.async_copy(deg_sh.at[ibuf.at[0]], rowbuf, sem).wait()
        pltpu.sync_copy(rowbuf,
                        cnt_hbm.at[pl.ds(c * npad + row0 + t * K, K)])
        return carry

    lax.fori_loop(0, rpt // K, rback, 0)


def _scat_body(npad, d, ept, hp_hbm, src_hbm, dst_hbm, acc_hbm,
               sibuf, dibuf, ibuf, zrows, rows, sem0, sem1,
               gsem, acc_sh):
    c = lax.axis_index("c")
    s = lax.axis_index("s")
    rpt = npad // NS
    row0 = s * rpt

    def zrow(r, carry):
        for j in range(d // L):
            zrows[r, pl.ds(j * L, L)] = jnp.zeros((L,), jnp.float32)
        return carry

    lax.fori_loop(0, K, zrow, 0)

    def iinit(t, carry):
        _fill_iota_idx(ibuf, 0, row0 + t * K)
        pltpu.sync_copy(zrows, acc_sh.at[ibuf.at[0]])
        return carry

    lax.fori_loop(0, rpt // K, iinit, 0)
    plsc.subcore_barrier()

    w = c * NS + s
    cpt = ept // K
    sems = (sem0, sem1)

    def batch(b, carry):
        pltpu.sync_copy(src_hbm.at[w, b], sibuf)
        pltpu.sync_copy(dst_hbm.at[w, b], dibuf)
        pltpu.async_copy(hp_hbm.at[sibuf.at[0]], rows.at[0], sem0)

        def step(k, c2):
            for j in range(2):
                ch = 2 * k + j
                pltpu.async_copy(hp_hbm.at[sibuf.at[ch + 1]], rows.at[1 - j],
                                 sems[1 - j])
                pltpu.make_async_copy(hp_hbm.at[sibuf.at[ch]], rows.at[j],
                                      sems[j]).wait()
                pltpu.sync_copy(rows.at[j], acc_sh.at[dibuf.at[ch]],
                                add=True)
            return c2

        lax.fori_loop(0, (NB - 1) // 2, step, 0)
        pltpu.make_async_copy(hp_hbm.at[sibuf.at[NB - 1]], rows.at[0],
                              sem0).wait()
        pltpu.sync_copy(rows.at[0], acc_sh.at[dibuf.at[NB - 1]], add=True)
        return carry

    lax.fori_loop(0, cpt // NB, batch, 0)

    plsc.subcore_barrier()

    def rback(t, carry):
        _fill_iota_idx(ibuf, 0, row0 + t * K)
        pltpu.async_copy(acc_sh.at[ibuf.at[0]], zrows, gsem).wait()
        pltpu.sync_copy(zrows,
                        acc_hbm.at[pl.ds(c * npad + row0 + t * K, K)])
        return carry

    lax.fori_loop(0, rpt // K, rback, 0)


def _mm_body(x_ref, w_ref, cnt0_ref, cnt1_ref, hp_ref):
    deg = 1.0 + cnt0_ref[:, 0:1] + cnt1_ref[:, 0:1]
    hp_ref[...] = jnp.dot(x_ref[...], w_ref[...],
                          preferred_element_type=jnp.float32) * lax.rsqrt(deg)


def _fin_body(acc0_ref, acc1_ref, hp_ref, cnt0_ref, cnt1_ref, b_ref,
              out_ref):
    deg = 1.0 + cnt0_ref[:, 0:1] + cnt1_ref[:, 0:1]
    out_ref[...] = (acc0_ref[...] + acc1_ref[...] + hp_ref[...]) \
        * lax.rsqrt(deg) + b_ref[...]


def kernel(x, edge_index, W, b):
    n, d_in = x.shape
    d_out = W.shape[1]
    e = edge_index.shape[1]
    nw = NC * NS
    ept = e // nw
    R = 1024
    npad = ((n + R - 1) // R) * R

    cpt = ept // K
    nbat = cpt // NB
    src = edge_index[0].reshape(nw, nbat, NB, K)
    dst = edge_index[1].reshape(nw, nbat, NB, K)

    mesh = plsc.VectorSubcoreMesh(core_axis_name="c", subcore_axis_name="s",
                                  num_cores=NC, num_subcores=NS)

    deg_call = pl.kernel(
        functools.partial(_deg_body, npad, ept),
        out_type=jax.ShapeDtypeStruct((NC * npad, L), jnp.float32),
        mesh=mesh,
        scratch_types=[
            pltpu.VMEM((1, K), jnp.int32),
            pltpu.VMEM((NB, K), jnp.int32),
            pltpu.VMEM((K, L), jnp.float32),
            pltpu.VMEM((K, L), jnp.float32),
            pltpu.SemaphoreType.DMA,
            pltpu.VMEM_SHARED((npad, L), jnp.float32),
        ],
    )
    cnt = deg_call(dst)
    cnt0 = cnt[:npad]
    cnt1 = cnt[npad:]

    grid = (npad // R,)
    blk = lambda i: (i, 0)
    hp = pl.pallas_call(
        _mm_body,
        grid=grid,
        in_specs=[
            pl.BlockSpec((R, d_in), blk),
            pl.BlockSpec((d_in, d_out), lambda i: (0, 0)),
            pl.BlockSpec((R, L), blk),
            pl.BlockSpec((R, L), blk),
        ],
        out_specs=pl.BlockSpec((R, d_out), blk),
        out_shape=jax.ShapeDtypeStruct((n, d_out), jnp.float32),
    )(x, W, cnt0, cnt1)

    scat_call = pl.kernel(
        functools.partial(_scat_body, npad, d_out, ept),
        out_type=jax.ShapeDtypeStruct((NC * npad, d_out), jnp.float32),
        mesh=mesh,
        scratch_types=[
            pltpu.VMEM((NB, K), jnp.int32),
            pltpu.VMEM((NB, K), jnp.int32),
            pltpu.VMEM((1, K), jnp.int32),
            pltpu.VMEM((K, d_out), jnp.float32),
            pltpu.VMEM((2, K, d_out), jnp.float32),
            pltpu.SemaphoreType.DMA,
            pltpu.SemaphoreType.DMA,
            pltpu.SemaphoreType.DMA,
            pltpu.VMEM_SHARED((npad, d_out), jnp.float32),
        ],
    )
    acc = scat_call(hp, src, dst)

    out = pl.pallas_call(
        _fin_body,
        grid=grid,
        in_specs=[
            pl.BlockSpec((R, d_out), blk),
            pl.BlockSpec((R, d_out), blk),
            pl.BlockSpec((R, d_out), blk),
            pl.BlockSpec((R, L), blk),
            pl.BlockSpec((R, L), blk),
            pl.BlockSpec((1, d_out), lambda i: (0, 0)),
        ],
        out_specs=pl.BlockSpec((R, d_out), blk),
        out_shape=jax.ShapeDtypeStruct((n, d_out), jnp.float32),
    )(acc[:npad], acc[npad:], hp, cnt0, cnt1, b.reshape(1, d_out))

    return out
